# baseline (speedup 1.0000x reference)
import functools

import jax
import jax.numpy as jnp
from jax import lax
from jax.experimental import pallas as pl
from jax.experimental.pallas import tpu as pltpu
from jax.experimental.pallas import tpu_sc as plsc

N_E = 8192
HALF = N_E // 2
D = 32
N_TOK = 16384
BT = 256
NBLK = N_TOK // BT
COMMIT = 0.25

NC = 2
NS = 16
NW = NC * NS
TOK_W = N_TOK // NW
SUB = 256
GCH = 128


def _bf16_rne(v):
    u = lax.bitcast_convert_type(v, jnp.uint32)
    lsb = (u >> jnp.uint32(16)) & jnp.uint32(1)
    r = (u + jnp.uint32(0x7FFF) + lsb) & jnp.uint32(0xFFFF0000)
    return lax.bitcast_convert_type(r, jnp.float32)


def _vq_body(x_ref, et_ref, xsq_ref, esq_ref,
             enc_ref, idx_ref, loss_ref, perp_ref,
             counts_acc, sse_acc):
    i = pl.program_id(0)
    mm = lax.dot_general(x_ref[...].astype(jnp.bfloat16),
                         et_ref[...].astype(jnp.bfloat16),
                         (((1,), (0,)), ((), ())),
                         preferred_element_type=jnp.float32)
    d = xsq_ref[...] + esq_ref[...] - 2.0 * mm
    cols = lax.broadcasted_iota(jnp.int32, (BT, N_E), 1)

    dl = d[:, :HALF]
    dr = d[:, HALF:]
    v1 = jnp.min(dl, axis=1, keepdims=True)
    c1 = jnp.min(jnp.where(dl == v1, cols[:, :HALF], N_E), axis=1, keepdims=True)
    v2 = jnp.min(dr, axis=1, keepdims=True)
    c2 = jnp.min(jnp.where(dr == v2, cols[:, HALF:], N_E), axis=1, keepdims=True)

    win_left = _bf16_rne(v1) <= v2
    idx = jnp.where(win_left, c1, c2)
    chosen_v = jnp.where(win_left, v1, v2)
    idx_ref[...] = idx

    cols16 = lax.broadcasted_iota(jnp.int16, (BT, N_E), 1)
    one_hot_bf = (cols16 == idx.astype(jnp.int16)).astype(jnp.bfloat16)
    enc_ref[...] = one_hot_bf.astype(jnp.float32)

    @pl.when(i == 0)
    def _init():
        counts_acc[...] = jnp.zeros_like(counts_acc)
        sse_acc[0] = 0.0

    counts_acc[...] += lax.dot_general(
        jnp.ones((1, BT), jnp.bfloat16), one_hot_bf,
        (((1,), (0,)), ((), ())), preferred_element_type=jnp.float32)
    sse_acc[0] += jnp.sum(chosen_v)

    @pl.when(i == NBLK - 1)
    def _fini():
        p = counts_acc[...] / jnp.float32(N_TOK)
        ent = jnp.sum(p * jnp.log(p + 1e-10), axis=1, keepdims=True)
        perp_ref[...] = jnp.exp(-ent)
        m = sse_acc[0] / jnp.float32(N_TOK * D)
        loss_ref[...] = jnp.full((1, 1), m + COMMIT * m, jnp.float32)


_vq_call = pl.pallas_call(
    _vq_body,
    grid=(NBLK,),
    in_specs=[
        pl.BlockSpec((BT, D), lambda i: (i, 0)),
        pl.BlockSpec((D, N_E), lambda i: (0, 0)),
        pl.BlockSpec((BT, 1), lambda i: (i, 0)),
        pl.BlockSpec((1, N_E), lambda i: (0, 0)),
    ],
    out_specs=[
        pl.BlockSpec((BT, N_E), lambda i: (i, 0)),
        pl.BlockSpec((BT, 1), lambda i: (i, 0)),
        pl.BlockSpec((1, 1), lambda i: (0, 0)),
        pl.BlockSpec((1, 1), lambda i: (0, 0)),
    ],
    out_shape=[
        jax.ShapeDtypeStruct((N_TOK, N_E), jnp.float32),
        jax.ShapeDtypeStruct((N_TOK, 1), jnp.int32),
        jax.ShapeDtypeStruct((1, 1), jnp.float32),
        jax.ShapeDtypeStruct((1, 1), jnp.float32),
    ],
    scratch_shapes=[
        pltpu.VMEM((1, N_E), jnp.float32),
        pltpu.SMEM((1,), jnp.float32),
    ],
    compiler_params=pltpu.CompilerParams(
        dimension_semantics=("arbitrary",),
    ),
)


_sc_mesh = plsc.VectorSubcoreMesh(core_axis_name="c", subcore_axis_name="s")


@functools.partial(
    pl.kernel,
    mesh=_sc_mesh,
    out_type=jax.ShapeDtypeStruct((N_TOK, D), jnp.float32),
    scratch_types=[
        pltpu.VMEM((TOK_W,), jnp.int32),
        pltpu.VMEM((SUB, 128), jnp.float32),
        pltpu.VMEM((SUB, D), jnp.float32),
        pltpu.SemaphoreType.DMA,
    ],
)
def _sc_gather_st(idx_hbm, table_hbm, x_hbm, out_hbm, idx_v, rows_v, x_v, sem):
    wid = lax.axis_index("s") * NC + lax.axis_index("c")
    base = wid * TOK_W
    pltpu.sync_copy(idx_hbm.at[pl.ds(base, TOK_W)], idx_v)
    for p in range(TOK_W // SUB):
        bp = base + p * SUB
        pltpu.sync_copy(x_hbm.at[pl.ds(bp, SUB)], x_v)
        copies = []
        for k in range(SUB // GCH):
            copies.append(pltpu.async_copy(
                table_hbm.at[idx_v.at[pl.ds(p * SUB + k * GCH, GCH)]],
                rows_v.at[pl.ds(k * GCH, GCH)], sem))
        for c in copies:
            c.wait()

        def body(r, carry):
            for h in range(D // 16):
                q16 = rows_v[r, pl.ds(h * 16, 16)]
                x16 = x_v[r, pl.ds(h * 16, 16)]
                x_v[r, pl.ds(h * 16, 16)] = x16 + (q16 - x16)
            return carry

        lax.fori_loop(0, SUB, body, 0)
        pltpu.sync_copy(x_v, out_hbm.at[pl.ds(bp, SUB)])


def kernel(inputs, embedding_weight):
    x = jnp.transpose(inputs, (0, 2, 3, 1))
    input_shape = x.shape
    flat = x.reshape(-1, D)
    xsq = jnp.sum(flat ** 2, axis=1, keepdims=True)
    esq = jnp.sum(embedding_weight ** 2, axis=1).reshape(1, N_E)
    et = embedding_weight.T
    enc, idxo, loss, perp = _vq_call(flat, et, xsq, esq)
    table = jnp.pad(embedding_weight.astype(jnp.bfloat16).astype(jnp.float32),
                    ((0, 0), (0, 128 - D)))
    qst = _sc_gather_st(idxo.reshape(N_TOK), table, flat)
    quantized_st = jnp.transpose(qst.reshape(input_shape), (0, 3, 1, 2))
    return (loss[0, 0], quantized_st, perp[0, 0], enc)


# speedup vs baseline: 1.4164x; 1.4164x over previous
import functools

import jax
import jax.numpy as jnp
from jax import lax
from jax.experimental import pallas as pl
from jax.experimental.pallas import tpu as pltpu
from jax.experimental.pallas import tpu_sc as plsc

N_E = 8192
HALF = N_E // 2
D = 32
N_TOK = 16384
BT = 256
NBLK = N_TOK // BT
COMMIT = 0.25

NC = 2
NS = 16
NW = NC * NS
TOK_W = N_TOK // NW
SUB = 256
GCH = 128


def _bf16_rne(v):
    u = lax.bitcast_convert_type(v, jnp.uint32)
    lsb = (u >> jnp.uint32(16)) & jnp.uint32(1)
    r = (u + jnp.uint32(0x7FFF) + lsb) & jnp.uint32(0xFFFF0000)
    return lax.bitcast_convert_type(r, jnp.float32)


def _vq_body(x_ref, et_ref, xsq_ref, esq_ref,
             enc_ref, idx_ref, loss_ref, perp_ref,
             counts_acc, sse_acc):
    i = pl.program_id(0)
    mm = lax.dot_general(x_ref[...].astype(jnp.bfloat16),
                         et_ref[...].astype(jnp.bfloat16),
                         (((1,), (0,)), ((), ())),
                         preferred_element_type=jnp.float32)
    d = xsq_ref[...] + esq_ref[...] - 2.0 * mm
    cols = lax.broadcasted_iota(jnp.int32, (BT, N_E), 1)

    dl = d[:, :HALF]
    dr = d[:, HALF:]
    v1 = jnp.min(dl, axis=1, keepdims=True)
    c1 = jnp.min(jnp.where(dl == v1, cols[:, :HALF], N_E), axis=1, keepdims=True)
    v2 = jnp.min(dr, axis=1, keepdims=True)
    c2 = jnp.min(jnp.where(dr == v2, cols[:, HALF:], N_E), axis=1, keepdims=True)

    win_left = _bf16_rne(v1) <= v2
    idx = jnp.where(win_left, c1, c2)
    chosen_v = jnp.where(win_left, v1, v2)
    idx_ref[...] = idx

    one_hot_bf = (cols == idx).astype(jnp.bfloat16)
    enc_ref[...] = one_hot_bf.astype(jnp.float32)

    @pl.when(i == 0)
    def _init():
        counts_acc[...] = jnp.zeros_like(counts_acc)
        sse_acc[0] = 0.0

    counts_acc[...] += lax.dot_general(
        jnp.ones((1, BT), jnp.bfloat16), one_hot_bf,
        (((1,), (0,)), ((), ())), preferred_element_type=jnp.float32)
    sse_acc[0] += jnp.sum(chosen_v)

    @pl.when(i == NBLK - 1)
    def _fini():
        p = counts_acc[...] / jnp.float32(N_TOK)
        ent = jnp.sum(p * jnp.log(p + 1e-10), axis=1, keepdims=True)
        perp_ref[...] = jnp.exp(-ent)
        m = sse_acc[0] / jnp.float32(N_TOK * D)
        loss_ref[...] = jnp.full((1, 1), m + COMMIT * m, jnp.float32)


_vq_call = pl.pallas_call(
    _vq_body,
    grid=(NBLK,),
    in_specs=[
        pl.BlockSpec((BT, D), lambda i: (i, 0)),
        pl.BlockSpec((D, N_E), lambda i: (0, 0)),
        pl.BlockSpec((BT, 1), lambda i: (i, 0)),
        pl.BlockSpec((1, N_E), lambda i: (0, 0)),
    ],
    out_specs=[
        pl.BlockSpec((BT, N_E), lambda i: (i, 0)),
        pl.BlockSpec((BT, 1), lambda i: (i, 0)),
        pl.BlockSpec((1, 1), lambda i: (0, 0)),
        pl.BlockSpec((1, 1), lambda i: (0, 0)),
    ],
    out_shape=[
        jax.ShapeDtypeStruct((N_TOK, N_E), jnp.float32),
        jax.ShapeDtypeStruct((N_TOK, 1), jnp.int32),
        jax.ShapeDtypeStruct((1, 1), jnp.float32),
        jax.ShapeDtypeStruct((1, 1), jnp.float32),
    ],
    scratch_shapes=[
        pltpu.VMEM((1, N_E), jnp.float32),
        pltpu.SMEM((1,), jnp.float32),
    ],
    compiler_params=pltpu.CompilerParams(
        dimension_semantics=("arbitrary",),
    ),
)


_sc_mesh = plsc.VectorSubcoreMesh(core_axis_name="c", subcore_axis_name="s")


@functools.partial(
    pl.kernel,
    mesh=_sc_mesh,
    out_type=jax.ShapeDtypeStruct((N_TOK, D), jnp.float32),
    scratch_types=[
        pltpu.VMEM((TOK_W,), jnp.int32),
        pltpu.VMEM((SUB, 128), jnp.float32),
        pltpu.VMEM((SUB, D), jnp.float32),
        pltpu.SemaphoreType.DMA,
    ],
)
def _sc_gather_st(idx_hbm, table_hbm, x_hbm, out_hbm, idx_v, rows_v, x_v, sem):
    wid = lax.axis_index("s") * NC + lax.axis_index("c")
    base = wid * TOK_W
    pltpu.sync_copy(idx_hbm.at[pl.ds(base, TOK_W)], idx_v)
    for p in range(TOK_W // SUB):
        bp = base + p * SUB
        pltpu.sync_copy(x_hbm.at[pl.ds(bp, SUB)], x_v)
        copies = []
        for k in range(SUB // GCH):
            copies.append(pltpu.async_copy(
                table_hbm.at[idx_v.at[pl.ds(p * SUB + k * GCH, GCH)]],
                rows_v.at[pl.ds(k * GCH, GCH)], sem))
        for c in copies:
            c.wait()

        def body(r, carry):
            for h in range(D // 16):
                q16 = rows_v[r, pl.ds(h * 16, 16)]
                x16 = x_v[r, pl.ds(h * 16, 16)]
                x_v[r, pl.ds(h * 16, 16)] = x16 + (q16 - x16)
            return carry

        lax.fori_loop(0, SUB, body, 0)
        pltpu.sync_copy(x_v, out_hbm.at[pl.ds(bp, SUB)])


def kernel(inputs, embedding_weight):
    x = jnp.transpose(inputs, (0, 2, 3, 1))
    input_shape = x.shape
    flat = x.reshape(-1, D)
    xsq = jnp.sum(flat ** 2, axis=1, keepdims=True)
    esq = jnp.sum(embedding_weight ** 2, axis=1).reshape(1, N_E)
    et = embedding_weight.T
    enc, idxo, loss, perp = _vq_call(flat, et, xsq, esq)
    table = jnp.pad(embedding_weight.astype(jnp.bfloat16).astype(jnp.float32),
                    ((0, 0), (0, 128 - D)))
    qst = _sc_gather_st(idxo.reshape(N_TOK), table, flat)
    quantized_st = jnp.transpose(qst.reshape(input_shape), (0, 3, 1, 2))
    return (loss[0, 0], quantized_st, perp[0, 0], enc)
